# Initial kernel scaffold; baseline (speedup 1.0000x reference)
import jax
import jax.numpy as jnp
from jax.experimental import pallas as pl


def kernel(logits, input_ids, attention_mask, z_t, t):
    raise NotImplementedError("write your pallas kernel here")


# speedup vs baseline: 5.7817x; 5.7817x over previous
import functools

import jax
import jax.numpy as jnp
from jax.experimental import pallas as pl
from jax.experimental.pallas import tpu as pltpu

VOCAB_MASK_ID = 1
NEG_VAL = -1000000.0
EPS_T = 0.0001


def _mdlm_body(nr_blocks, nv_blocks, r_blk, v_blk, s_len,
               logits_ref, ids_ref, z_ref, attn_ref, t_ref,
               elbo_ref, loss_ref,
               m_ref, s_ref, xg_ref, acc_ref):
    i = pl.program_id(0)
    j = pl.program_id(1)

    x = logits_ref[...]
    col0 = j * v_blk
    local_cols = jax.lax.broadcasted_iota(jnp.int32, (1, v_blk), 1)
    xm = jnp.where(local_cols == (VOCAB_MASK_ID - col0), NEG_VAL, x)

    bm = jnp.max(xm, axis=1, keepdims=True)
    bs = jnp.sum(jnp.exp(xm - bm), axis=1, keepdims=True)

    ids = ids_ref[...]
    hit = (local_cols == (ids - col0))
    bx = jnp.sum(jnp.where(hit, xm, 0.0), axis=1, keepdims=True)

    @pl.when(j == 0)
    def _init():
        m_ref[...] = bm
        s_ref[...] = bs
        xg_ref[...] = bx

    @pl.when(j > 0)
    def _merge():
        m_old = m_ref[...]
        s_old = s_ref[...]
        m_new = jnp.maximum(m_old, bm)
        s_ref[...] = s_old * jnp.exp(m_old - m_new) + bs * jnp.exp(bm - m_new)
        m_ref[...] = m_new
        xg_ref[...] = xg_ref[...] + bx

    @pl.when(jnp.logical_and(i == 0, j == 0))
    def _init_acc():
        acc_ref[0] = 0.0
        acc_ref[1] = 0.0

    @pl.when(j == nv_blocks - 1)
    def _epilogue():
        lse = m_ref[...] + jnp.log(s_ref[...])
        xg = xg_ref[...]
        maskf = (z_ref[...] == VOCAB_MASK_ID).astype(jnp.float32)
        b = (i * r_blk) // s_len
        w = 1.0 / jnp.clip(t_ref[b], EPS_T, 1.0)
        elbo = maskf * w * (lse - xg)
        elbo_ref[...] = elbo
        attn = attn_ref[...]
        acc_ref[0] = acc_ref[0] + jnp.sum(elbo * attn)
        acc_ref[1] = acc_ref[1] + jnp.sum(attn)

        @pl.when(i == nr_blocks - 1)
        def _final():
            loss_ref[0, 0] = acc_ref[0] / acc_ref[1]


def kernel(logits, input_ids, attention_mask, z_t, t):
    B, S, V = logits.shape
    rows = B * S

    v_blk = 6400 if V % 6400 == 0 else V
    r_blk = 256 if (rows % 256 == 0 and S % 256 == 0) else S
    nr_blocks = rows // r_blk
    nv_blocks = V // v_blk

    logits2 = logits.reshape(rows, V)
    ids2 = input_ids.astype(jnp.int32).reshape(rows, 1)
    z2 = z_t.astype(jnp.int32).reshape(rows, 1)
    attn2 = attention_mask.astype(jnp.float32).reshape(rows, 1)
    t1 = t.astype(jnp.float32)

    body = functools.partial(_mdlm_body, nr_blocks, nv_blocks, r_blk, v_blk, S)

    elbo_flat, loss11 = pl.pallas_call(
        body,
        grid=(nr_blocks, nv_blocks),
        in_specs=[
            pl.BlockSpec((r_blk, v_blk), lambda i, j: (i, j)),
            pl.BlockSpec((r_blk, 1), lambda i, j: (i, 0)),
            pl.BlockSpec((r_blk, 1), lambda i, j: (i, 0)),
            pl.BlockSpec((r_blk, 1), lambda i, j: (i, 0)),
            pl.BlockSpec(memory_space=pltpu.SMEM),
        ],
        out_specs=[
            pl.BlockSpec((r_blk, 1), lambda i, j: (i, 0)),
            pl.BlockSpec(memory_space=pltpu.SMEM),
        ],
        out_shape=[
            jax.ShapeDtypeStruct((rows, 1), jnp.float32),
            jax.ShapeDtypeStruct((1, 1), jnp.float32),
        ],
        scratch_shapes=[
            pltpu.VMEM((r_blk, 1), jnp.float32),
            pltpu.VMEM((r_blk, 1), jnp.float32),
            pltpu.VMEM((r_blk, 1), jnp.float32),
            pltpu.SMEM((2,), jnp.float32),
        ],
        compiler_params=pltpu.CompilerParams(
            dimension_semantics=("arbitrary", "arbitrary"),
        ),
    )(logits2, ids2, z2, attn2, t1)

    loss = loss11[0, 0]
    elbo = elbo_flat[:, 0].reshape(B, S)
    return (loss, elbo, loss, loss)
